# baseline (speedup 1.0000x reference)
import functools

import jax
import jax.numpy as jnp
from jax import lax
from jax.experimental import pallas as pl
from jax.experimental.pallas import tpu as pltpu
from jax.experimental.pallas import tpu_sc as plsc

_B = 16384
_D = 128
_NC, _NS, _L = 2, 16, 16
_NW = _NC * _NS
_RPW = _B // _NW
_HALF = _RPW // 2
_GH = _HALF // _L


def _half(z_ref, g_v, y2_v, out_v, rbase):
    lanes = lax.iota(jnp.int32, _L)

    def group(gi):
        lrow = gi * _L + lanes
        crow = rbase + lrow
        idxvec = plsc.load_gather(y2_v, [crow * 2])
        jv = crow & (_D - 1)
        acc = [jnp.zeros((_L,), jnp.float32) for _ in range(4)]
        for j in range(_D):
            zc = plsc.load_gather(z_ref, [lrow, jv])
            gc = plsc.load_gather(g_v, [idxvec, jv])
            acc[j % 4] = acc[j % 4] + zc * gc
            jv = (jv + 1) & (_D - 1)
        out_v[pl.ds(rbase + gi * _L, _L)] = (acc[0] + acc[1]) + (acc[2] + acc[3])

    plsc.parallel_loop(0, _GH, 1)(group)


def _sc_body(z_hbm, y_hbm, g_hbm, out_hbm,
             z0_v, z1_v, g_v, y2_v, out_v, sem_g, sem_y, sem_z0, sem_z1):
    wid = lax.axis_index("s") * _NC + lax.axis_index("c")
    base = wid * _RPW
    cp_g = pltpu.async_copy(g_hbm, g_v, sem_g)
    cp_y = pltpu.async_copy(y_hbm.at[pl.ds(base * 2, _RPW * 2)], y2_v, sem_y)
    cp_z0 = pltpu.async_copy(z_hbm.at[pl.ds(base, _HALF)], z0_v, sem_z0)
    cp_z1 = pltpu.async_copy(z_hbm.at[pl.ds(base + _HALF, _HALF)], z1_v, sem_z1)
    cp_g.wait()
    cp_y.wait()
    cp_z0.wait()
    _half(z0_v, g_v, y2_v, out_v, 0)
    cp_z1.wait()
    _half(z1_v, g_v, y2_v, out_v, _HALF)
    pltpu.sync_copy(out_v, out_hbm.at[pl.ds(base, _RPW)])


@functools.cache
def _sc_call():
    return functools.partial(
        pl.kernel,
        out_type=jax.ShapeDtypeStruct((_B,), jnp.float32),
        mesh=plsc.VectorSubcoreMesh(
            core_axis_name="c", subcore_axis_name="s",
            num_cores=_NC, num_subcores=_NS),
        compiler_params=pltpu.CompilerParams(needs_layout_passes=False),
        scratch_types=[
            pltpu.VMEM((_HALF, _D), jnp.float32),
            pltpu.VMEM((_HALF, _D), jnp.float32),
            pltpu.VMEM((4, _D), jnp.float32),
            pltpu.VMEM((_RPW * 2,), jnp.int32),
            pltpu.VMEM((_RPW,), jnp.float32),
            pltpu.SemaphoreType.DMA,
            pltpu.SemaphoreType.DMA,
            pltpu.SemaphoreType.DMA,
            pltpu.SemaphoreType.DMA,
        ],
    )(_sc_body)


def kernel(z, y, gamma):
    return _sc_call()(z, y.astype(jnp.int32).reshape(-1), gamma)


# speedup vs baseline: 1.3097x; 1.3097x over previous
import functools

import jax
import jax.numpy as jnp
from jax import lax
from jax.experimental import pallas as pl
from jax.experimental.pallas import tpu as pltpu
from jax.experimental.pallas import tpu_sc as plsc

_B = 16384
_D = 128
_NC, _NS, _L = 2, 16, 16
_NW = _NC * _NS
_RPW = _B // _NW
_HALF = _RPW // 2
_GH = _HALF // _L


def _half(z_ref, g_v, y2_v, out_v, rbase):
    lanes = lax.iota(jnp.int32, _L)

    def group(gi):
        lrow = gi * _L + lanes
        crow = rbase + lrow
        idxvec = y2_v[pl.ds(rbase + gi * _L, _L)]
        jv = crow & (_D - 1)
        acc = [jnp.zeros((_L,), jnp.float32) for _ in range(4)]
        for j in range(_D):
            zc = plsc.load_gather(z_ref, [lrow, jv])
            gc = plsc.load_gather(g_v, [idxvec, jv])
            acc[j % 4] = acc[j % 4] + zc * gc
            jv = (jv + 1) & (_D - 1)
        out_v[pl.ds(rbase + gi * _L, _L)] = (acc[0] + acc[1]) + (acc[2] + acc[3])

    plsc.parallel_loop(0, _GH, 1)(group)


def _sc_body(z_hbm, y_hbm, g_hbm, out_hbm,
             z0_v, z1_v, g_v, y2_v, out_v, sem_g, sem_y, sem_z0, sem_z1):
    wid = lax.axis_index("s") * _NC + lax.axis_index("c")
    base = wid * _RPW
    cp_g = pltpu.async_copy(g_hbm, g_v, sem_g)
    cp_y = pltpu.async_copy(y_hbm.at[pl.ds(base, _RPW)], y2_v, sem_y)
    cp_z0 = pltpu.async_copy(z_hbm.at[pl.ds(base, _HALF)], z0_v, sem_z0)
    cp_z1 = pltpu.async_copy(z_hbm.at[pl.ds(base + _HALF, _HALF)], z1_v, sem_z1)
    cp_g.wait()
    cp_y.wait()
    cp_z0.wait()
    _half(z0_v, g_v, y2_v, out_v, 0)
    cp_z1.wait()
    _half(z1_v, g_v, y2_v, out_v, _HALF)
    pltpu.sync_copy(out_v, out_hbm.at[pl.ds(base, _RPW)])


@functools.cache
def _sc_call():
    return functools.partial(
        pl.kernel,
        out_type=jax.ShapeDtypeStruct((_B,), jnp.float32),
        mesh=plsc.VectorSubcoreMesh(
            core_axis_name="c", subcore_axis_name="s",
            num_cores=_NC, num_subcores=_NS),
        compiler_params=pltpu.CompilerParams(needs_layout_passes=False),
        scratch_types=[
            pltpu.VMEM((_HALF, _D), jnp.float32),
            pltpu.VMEM((_HALF, _D), jnp.float32),
            pltpu.VMEM((4, _D), jnp.float32),
            pltpu.VMEM((_RPW,), jnp.int32),
            pltpu.VMEM((_RPW,), jnp.float32),
            pltpu.SemaphoreType.DMA,
            pltpu.SemaphoreType.DMA,
            pltpu.SemaphoreType.DMA,
            pltpu.SemaphoreType.DMA,
        ],
    )(_sc_body)


def kernel(z, y, gamma):
    return _sc_call()(z, y[:, 0].astype(jnp.int32), gamma)
